# Initial kernel scaffold; baseline (speedup 1.0000x reference)
import jax
import jax.numpy as jnp
from jax.experimental import pallas as pl


def kernel(inputs, table, bias):
    raise NotImplementedError("write your pallas kernel here")


# speedup vs baseline: 1.0699x; 1.0699x over previous
import functools

import jax
import jax.numpy as jnp
import numpy as np
from jax import lax
from jax.experimental import pallas as pl
from jax.experimental.pallas import tpu as pltpu
from jax.experimental.pallas import tpu_sc as plsc

_CARD = 10000


def _sc_workers():
    try:
        info = plsc.get_sparse_core_info()
        return info.num_cores, info.num_subcores
    except Exception:
        return 2, 16


def _build(B, F, D, NC, NS):
    NW = NC * NS
    N = B * F
    C = 512
    NBLK = C // 128
    per_w = N // NW
    n_chunks = per_w // C
    ext_rows = ((F - 1 + C) // F + 1) * F
    off_len = ((F - 1 + C + 15) // 16 + 1) * 16

    mesh = plsc.VectorSubcoreMesh(core_axis_name="c", subcore_axis_name="s")

    @functools.partial(
        pl.kernel,
        mesh=mesh,
        compiler_params=pltpu.CompilerParams(use_tc_tiling_on_sc=False),
        out_type=jax.ShapeDtypeStruct((N, D), jnp.float32),
        scratch_types=[
            pltpu.VMEM((NBLK, 128), jnp.int32),
            pltpu.VMEM((C, D), jnp.float32),
            pltpu.VMEM((off_len,), jnp.int32),
            pltpu.VMEM((ext_rows * D,), jnp.float32),
            pltpu.SemaphoreType.DMA,
        ],
    )
    def run(idx_hbm, table_hbm, off_hbm, bias_hbm, out_hbm,
            idx_v, rows_v, off_v, bias_v, gsem):
        cid = lax.axis_index("c")
        sid = lax.axis_index("s")
        wid = sid * NC + cid
        base_blk = wid * (per_w // 128)

        pltpu.sync_copy(off_hbm, off_v)
        pltpu.sync_copy(bias_hbm, bias_v)

        @pl.loop(0, n_chunks)
        def _chunk(c):
            p = lax.rem(c * C, F)
            pltpu.sync_copy(idx_hbm.at[pl.ds(base_blk + c * NBLK, NBLK)],
                            idx_v)

            @plsc.parallel_loop(0, C // 16, unroll=4)
            def _adj(u):
                blk = u >> 3
                qoff = (u & 7) * 16
                idx_v[blk, pl.ds(qoff, 16)] += off_v[pl.ds(p + u * 16, 16)]

            descs = []
            for blk in range(NBLK):
                descs.append(pltpu.async_copy(
                    table_hbm.at[idx_v.at[blk]],
                    rows_v.at[pl.ds(blk * 128, 128)], gsem))
            for d in descs:
                d.wait()

            @plsc.parallel_loop(0, C, unroll=4)
            def _badd(r):
                boff = (p + r) * D
                rows_v[r, pl.ds(0, 16)] += bias_v[pl.ds(boff, 16)]
                rows_v[r, pl.ds(16, 16)] += bias_v[pl.ds(boff + 16, 16)]

            pltpu.sync_copy(rows_v,
                            out_hbm.at[pl.ds(wid * per_w + c * C, C)])

    return run


def kernel(inputs, table, bias):
    B, F = inputs.shape
    D = table.shape[1]
    NC, NS = _sc_workers()
    C = 512
    ext_rows = ((F - 1 + C) // F + 1) * F
    off_len = ((F - 1 + C + 15) // 16 + 1) * 16

    idx2d = inputs.astype(jnp.int32).reshape(B * F // 128, 128)
    off_host = jnp.asarray(
        (np.arange(off_len) % F) * _CARD, dtype=jnp.int32)
    bias_ext = jnp.tile(bias.astype(jnp.float32),
                        (ext_rows // F, 1)).reshape(-1)

    run = _build(B, F, D, NC, NS)
    out = run(idx2d, table, off_host, bias_ext)
    return out.reshape(B, F, D)


# speedup vs baseline: 1.1028x; 1.0308x over previous
import functools

import jax
import jax.numpy as jnp
import numpy as np
from jax import lax
from jax.experimental import pallas as pl
from jax.experimental.pallas import tpu as pltpu
from jax.experimental.pallas import tpu_sc as plsc

_CARD = 10000


def _sc_workers():
    try:
        info = plsc.get_sparse_core_info()
        return info.num_cores, info.num_subcores
    except Exception:
        return 2, 16


def _build(B, F, D, NC, NS):
    NW = NC * NS
    N = B * F
    C = 512
    NBLK = C // 128
    per_w = N // NW
    n_chunks = per_w // C
    ext_rows = ((F - 1 + C) // F + 1) * F
    off_len = ((F - 1 + C + 15) // 16 + 1) * 16

    mesh = plsc.VectorSubcoreMesh(core_axis_name="c", subcore_axis_name="s")

    NBUF = 4

    @functools.partial(
        pl.kernel,
        mesh=mesh,
        compiler_params=pltpu.CompilerParams(use_tc_tiling_on_sc=False),
        out_type=jax.ShapeDtypeStruct((N, D), jnp.float32),
        scratch_types=[
            pltpu.VMEM((NBUF, NBLK, 128), jnp.int32),
            pltpu.VMEM((NBUF, C, D), jnp.float32),
            pltpu.VMEM((off_len,), jnp.int32),
            pltpu.VMEM((ext_rows * D,), jnp.float32),
            [pltpu.SemaphoreType.DMA] * NBUF,
            [pltpu.SemaphoreType.DMA] * NBUF,
        ],
    )
    def run(idx_hbm, table_hbm, off_hbm, bias_hbm, out_hbm,
            idx_v, rows_v, off_v, bias_v, gsems, ssems):
        cid = lax.axis_index("c")
        sid = lax.axis_index("s")
        wid = sid * NC + cid
        base_blk = wid * (per_w // 128)

        pltpu.sync_copy(off_hbm, off_v)
        pltpu.sync_copy(bias_hbm, bias_v)

        def prep(c, b):
            p = lax.rem(c * C, F)
            pltpu.sync_copy(idx_hbm.at[pl.ds(base_blk + c * NBLK, NBLK)],
                            idx_v.at[b])

            @plsc.parallel_loop(0, C // 16, unroll=4)
            def _adj(u):
                blk = u >> 3
                qoff = (u & 7) * 16
                idx_v[b, blk, pl.ds(qoff, 16)] += off_v[pl.ds(p + u * 16, 16)]

            for blk in range(NBLK):
                pltpu.async_copy(table_hbm.at[idx_v.at[b, blk]],
                                 rows_v.at[b, pl.ds(blk * 128, 128)],
                                 gsems[b])

        def wait_gathers(b):
            pltpu.make_async_copy(table_hbm.at[pl.ds(0, C)],
                                  rows_v.at[b], gsems[b]).wait()

        def wait_store(b):
            pltpu.make_async_copy(rows_v.at[b],
                                  out_hbm.at[pl.ds(0, C)], ssems[b]).wait()

        def proc(c, b):
            wait_gathers(b)
            p = lax.rem(c * C, F)

            @plsc.parallel_loop(0, C, unroll=4)
            def _badd(r):
                boff = (p + r) * D
                rows_v[b, r, pl.ds(0, 16)] += bias_v[pl.ds(boff, 16)]
                rows_v[b, r, pl.ds(16, 16)] += bias_v[pl.ds(boff + 16, 16)]

            pltpu.async_copy(rows_v.at[b],
                             out_hbm.at[pl.ds(wid * per_w + c * C, C)],
                             ssems[b])

        prep(0, 0)
        prep(1, 1)

        @pl.loop(0, n_chunks // NBUF)
        def _macro(cc):
            c0 = cc * NBUF
            for j in range(NBUF):
                c = c0 + j
                proc(c, j)
                b2 = (j + 2) % NBUF

                @pl.when(c + 2 < n_chunks)
                def _():
                    @pl.when(c >= 2)
                    def _():
                        wait_store(b2)
                    prep(c + 2, b2)

        for b in range(NBUF):
            wait_store(b)

    return run


def kernel(inputs, table, bias):
    B, F = inputs.shape
    D = table.shape[1]
    NC, NS = _sc_workers()
    C = 512
    ext_rows = ((F - 1 + C) // F + 1) * F
    off_len = ((F - 1 + C + 15) // 16 + 1) * 16

    idx2d = inputs.astype(jnp.int32).reshape(B * F // 128, 128)
    off_host = jnp.asarray(
        (np.arange(off_len) % F) * _CARD, dtype=jnp.int32)
    bias_ext = jnp.tile(bias.astype(jnp.float32),
                        (ext_rows // F, 1)).reshape(-1)

    run = _build(B, F, D, NC, NS)
    out = run(idx2d, table, off_host, bias_ext)
    return out.reshape(B, F, D)


# speedup vs baseline: 3.3965x; 3.0798x over previous
import functools

import jax
import jax.numpy as jnp
from jax import lax
from jax.experimental import pallas as pl
from jax.experimental.pallas import tpu as pltpu
from jax.experimental.pallas import tpu_sc as plsc

_CARD = 10000


def _sc_workers():
    try:
        info = plsc.get_sparse_core_info()
        return info.num_cores, info.num_subcores
    except Exception:
        return 2, 16


def _build(B, F, D, NC, NS):
    NW = NC * NS
    NB = B // 128
    DT = D // 8
    n_tiles = F * NB
    G = 4
    n_groups_w = n_tiles // (NW * G)

    mesh = plsc.VectorSubcoreMesh(core_axis_name="c", subcore_axis_name="s")

    @functools.partial(
        pl.kernel,
        mesh=mesh,
        compiler_params=pltpu.CompilerParams(use_tc_tiling_on_sc=False,
                                             needs_layout_passes=False),
        out_type=jax.ShapeDtypeStruct((F, DT, NB, 8, 128), jnp.float32),
        scratch_types=[
            pltpu.VMEM((2, G, 128), jnp.int32),
            pltpu.VMEM((2, G, 128, D), jnp.float32),
            pltpu.VMEM((2, DT, G, 8, 128), jnp.float32),
            pltpu.VMEM((2, D, 16), jnp.float32),
            [pltpu.SemaphoreType.DMA] * 2,
            [pltpu.SemaphoreType.DMA] * 2,
        ],
    )
    def run(idx_hbm, table_hbm, bsp_hbm, out_hbm,
            idx_v, rows_v, outb_v, bsp_v, gsems, ssems):
        cid = lax.axis_index("c")
        sid = lax.axis_index("s")
        wid = sid * NC + cid
        t_base = wid * n_groups_w * G

        def prep(g, q):
            t0 = t_base + g * G
            f = lax.div(t0, NB)
            bt0 = lax.rem(t0, NB)
            pltpu.sync_copy(idx_hbm.at[f, pl.ds(bt0, G)], idx_v.at[q])
            pltpu.sync_copy(bsp_hbm.at[f], bsp_v.at[q])
            off = jnp.full((16,), f * _CARD, dtype=jnp.int32)

            @plsc.parallel_loop(0, G * 8, unroll=4)
            def _adj(u):
                idx_v[q, u >> 3, pl.ds((u & 7) * 16, 16)] += off

            for j in range(G):
                pltpu.async_copy(table_hbm.at[idx_v.at[q, j]],
                                 rows_v.at[q, j], gsems[q])

        def proc(g, q):
            for j in range(G):
                pltpu.make_async_copy(table_hbm.at[idx_v.at[q, j]],
                                      rows_v.at[q, j], gsems[q]).wait()

            qv = jnp.full((16,), q, dtype=jnp.int32)
            lanes = lax.iota(jnp.int32, 16)

            @pl.loop(0, G * 8)
            def _tp(u):
                j = u >> 3
                bs0 = (u & 7) * 16
                jv = jnp.full((16,), j, dtype=jnp.int32)
                ib = lanes + bs0
                for d in range(D):
                    v = plsc.load_gather(
                        rows_v, [qv, jv, ib,
                                 jnp.full((16,), d, dtype=jnp.int32)])
                    outb_v[q, d >> 3, j, d & 7, pl.ds(bs0, 16)] = (
                        v + bsp_v[q, d])

            t0 = t_base + g * G
            f = lax.div(t0, NB)
            bt0 = lax.rem(t0, NB)
            pltpu.async_copy(outb_v.at[q],
                             out_hbm.at[f, :, pl.ds(bt0, G)], ssems[q])

        def wait_store(q):
            pltpu.make_async_copy(outb_v.at[q],
                                  out_hbm.at[0, :, pl.ds(0, G)],
                                  ssems[q]).wait()

        prep(0, 0)

        @pl.loop(0, n_groups_w // 2)
        def _main(gg):
            g0 = gg * 2
            prep(g0 + 1, 1)

            @pl.when(g0 >= 2)
            def _():
                wait_store(0)
            proc(g0, 0)

            @pl.when(g0 + 2 < n_groups_w)
            def _():
                prep(g0 + 2, 0)

            @pl.when(g0 >= 1)
            def _():
                wait_store(1)
            proc(g0 + 1, 1)

        wait_store(0)
        wait_store(1)

    return run


def kernel(inputs, table, bias):
    B, F = inputs.shape
    D = table.shape[1]
    NC, NS = _sc_workers()
    NB = B // 128
    DT = D // 8

    idx_t = inputs.astype(jnp.int32).T.reshape(F, NB, 128)
    bsp = jnp.broadcast_to(bias.astype(jnp.float32)[:, :, None], (F, D, 16))

    run = _build(B, F, D, NC, NS)
    out5d = run(idx_t, table, bsp)
    out = out5d.transpose(2, 4, 0, 1, 3).reshape(B, F, D)
    return out


# speedup vs baseline: 5.3194x; 1.5661x over previous
import functools

import jax
import jax.numpy as jnp
from jax import lax
from jax.experimental import pallas as pl
from jax.experimental.pallas import tpu as pltpu
from jax.experimental.pallas import tpu_sc as plsc

_CARD = 10000


def _sc_workers():
    try:
        info = plsc.get_sparse_core_info()
        return info.num_cores, info.num_subcores
    except Exception:
        return 2, 16


def _build(B, F, D, NC, NS):
    NW = NC * NS
    NB = B // 128
    DT = D // 8
    n_tiles = F * NB
    G = 4
    n_groups_w = n_tiles // (NW * G)
    GBLK = DT * G * 8 * 128

    mesh = plsc.VectorSubcoreMesh(core_axis_name="c", subcore_axis_name="s")

    @functools.partial(
        pl.kernel,
        mesh=mesh,
        compiler_params=pltpu.CompilerParams(use_tc_tiling_on_sc=False,
                                             needs_layout_passes=False),
        out_type=jax.ShapeDtypeStruct((F * DT * NB * 8 * 128,), jnp.float32),
        scratch_types=[
            pltpu.VMEM((2, G, 128), jnp.int32),
            pltpu.VMEM((2, G, 128, D), jnp.float32),
            pltpu.VMEM((2 * GBLK,), jnp.float32),
            pltpu.VMEM((F, D), jnp.float32),
            [pltpu.SemaphoreType.DMA] * 2,
            [pltpu.SemaphoreType.DMA] * 2,
        ],
    )
    def run(idx_hbm, table_hbm, bias_hbm, out_hbm,
            idx_v, rows_v, outb_v, bias_v, gsems, ssems):
        cid = lax.axis_index("c")
        sid = lax.axis_index("s")
        wid = sid * NC + cid
        t_base = wid * n_groups_w * G

        pltpu.sync_copy(bias_hbm, bias_v)

        def prep(g, q):
            t0 = t_base + g * G
            f = lax.div(t0, NB)
            bt0 = lax.rem(t0, NB)
            pltpu.sync_copy(idx_hbm.at[f, pl.ds(bt0, G)], idx_v.at[q])
            off = jnp.full((16,), f * _CARD, dtype=jnp.int32)

            @plsc.parallel_loop(0, G * 8, unroll=4)
            def _adj(u):
                idx_v[q, u >> 3, pl.ds((u & 7) * 16, 16)] += off

            for j in range(G):
                pltpu.async_copy(table_hbm.at[idx_v.at[q, j]],
                                 rows_v.at[q, j], gsems[q])

        def proc(g, q):
            for j in range(G):
                pltpu.make_async_copy(table_hbm.at[idx_v.at[q, j]],
                                      rows_v.at[q, j], gsems[q]).wait()

            t0 = t_base + g * G
            f = lax.div(t0, NB)
            bt0 = lax.rem(t0, NB)
            b0 = bias_v[f, pl.ds(0, 16)]
            b1 = bias_v[f, pl.ds(16, 16)]
            lanes = lax.iota(jnp.int32, 16)
            c0 = (lanes >> 3) * (G * 8 * 128) + (lanes & 7) * 128 + q * GBLK
            c1 = ((lanes + 16) >> 3) * (G * 8 * 128) \
                + ((lanes + 16) & 7) * 128 + q * GBLK

            @plsc.parallel_loop(0, G * 128, unroll=2)
            def _tp(rr):
                j = rr >> 7
                r = rr & 127
                s = jnp.full((16,), j * 1024 + r, dtype=jnp.int32)
                plsc.store_scatter(outb_v, [c0 + s],
                                   rows_v[q, j, r, pl.ds(0, 16)] + b0)
                plsc.store_scatter(outb_v, [c1 + s],
                                   rows_v[q, j, r, pl.ds(16, 16)] + b1)

            base = f * (DT * NB * 8 * 128) + bt0 * 1024
            for dt in range(DT):
                pltpu.async_copy(
                    outb_v.at[pl.ds(q * GBLK + dt * (G * 8 * 128),
                                    G * 8 * 128)],
                    out_hbm.at[pl.ds(base + dt * (NB * 8 * 128),
                                     G * 8 * 128)],
                    ssems[q])

        def wait_store(q):
            for dt in range(DT):
                pltpu.make_async_copy(
                    outb_v.at[pl.ds(q * GBLK, G * 8 * 128)],
                    out_hbm.at[pl.ds(0, G * 8 * 128)], ssems[q]).wait()

        prep(0, 0)

        @pl.loop(0, n_groups_w // 2)
        def _main(gg):
            g0 = gg * 2
            prep(g0 + 1, 1)

            @pl.when(g0 >= 2)
            def _():
                wait_store(0)
            proc(g0, 0)

            @pl.when(g0 + 2 < n_groups_w)
            def _():
                prep(g0 + 2, 0)

            @pl.when(g0 >= 1)
            def _():
                wait_store(1)
            proc(g0 + 1, 1)

        wait_store(0)
        wait_store(1)

    return run


def kernel(inputs, table, bias):
    B, F = inputs.shape
    D = table.shape[1]
    NC, NS = _sc_workers()
    NB = B // 128
    DT = D // 8

    idx_t = inputs.astype(jnp.int32).T.reshape(F, NB, 128)

    run = _build(B, F, D, NC, NS)
    out_flat = run(idx_t, table, bias.astype(jnp.float32))
    out = (out_flat.reshape(F, DT, NB, 8, 128)
           .transpose(2, 4, 0, 1, 3).reshape(B, F, D))
    return out


# speedup vs baseline: 5.5600x; 1.0452x over previous
import functools

import jax
import jax.numpy as jnp
from jax import lax
from jax.experimental import pallas as pl
from jax.experimental.pallas import tpu as pltpu
from jax.experimental.pallas import tpu_sc as plsc

_CARD = 10000


def _sc_workers():
    try:
        info = plsc.get_sparse_core_info()
        return info.num_cores, info.num_subcores
    except Exception:
        return 2, 16


def _build(B, F, D, NC, NS):
    NW = NC * NS
    NB = B // 128
    DT = D // 8
    n_tiles = F * NB
    G = 4
    n_groups_w = n_tiles // (NW * G)
    GBLK = DT * G * 8 * 128

    mesh = plsc.VectorSubcoreMesh(core_axis_name="c", subcore_axis_name="s")

    @functools.partial(
        pl.kernel,
        mesh=mesh,
        compiler_params=pltpu.CompilerParams(use_tc_tiling_on_sc=False,
                                             needs_layout_passes=False),
        out_type=jax.ShapeDtypeStruct((F * DT * NB * 8 * 128,), jnp.float32),
        scratch_types=[
            pltpu.VMEM((4, G, 128), jnp.int32),
            pltpu.VMEM((4, G, 128, D), jnp.float32),
            pltpu.VMEM((2 * GBLK,), jnp.float32),
            pltpu.VMEM((F, D), jnp.float32),
            [pltpu.SemaphoreType.DMA] * 4,
            [pltpu.SemaphoreType.DMA] * 4,
            [pltpu.SemaphoreType.DMA] * 2,
        ],
    )
    def run(idx_hbm, table_hbm, bias_hbm, out_hbm,
            idx_v, rows_v, outb_v, bias_v, isems, gsems, ssems):
        cid = lax.axis_index("c")
        sid = lax.axis_index("s")
        wid = sid * NC + cid
        t_base = wid * n_groups_w * G

        pltpu.sync_copy(bias_hbm, bias_v)

        def fkey(g):
            t0 = t_base + g * G
            return lax.div(t0, NB), lax.rem(t0, NB)

        def prep_a(g, p):
            f, bt0 = fkey(g)
            pltpu.async_copy(idx_hbm.at[f, pl.ds(bt0, G)], idx_v.at[p],
                             isems[p])

        def prep_b(g, p):
            pltpu.make_async_copy(idx_hbm.at[0, pl.ds(0, G)], idx_v.at[p],
                                  isems[p]).wait()
            f, _ = fkey(g)
            seg = table_hbm.at[pl.ds(f * _CARD, _CARD)]
            for j in range(G):
                pltpu.async_copy(seg.at[idx_v.at[p, j]],
                                 rows_v.at[p, j], gsems[p])

        def proc(g, p, o):
            for j in range(G):
                pltpu.make_async_copy(table_hbm.at[idx_v.at[p, j]],
                                      rows_v.at[p, j], gsems[p]).wait()

            f, bt0 = fkey(g)
            b0 = bias_v[f, pl.ds(0, 16)]
            b1 = bias_v[f, pl.ds(16, 16)]
            lanes = lax.iota(jnp.int32, 16)
            c0 = (lanes >> 3) * (G * 8 * 128) + (lanes & 7) * 128 + o * GBLK
            c1 = ((lanes + 16) >> 3) * (G * 8 * 128) \
                + ((lanes + 16) & 7) * 128 + o * GBLK

            @plsc.parallel_loop(0, G * 128, unroll=4)
            def _tp(rr):
                j = rr >> 7
                r = rr & 127
                s = jnp.full((16,), j * 1024 + r, dtype=jnp.int32)
                plsc.store_scatter(outb_v, [c0 + s],
                                   rows_v[p, j, r, pl.ds(0, 16)] + b0)
                plsc.store_scatter(outb_v, [c1 + s],
                                   rows_v[p, j, r, pl.ds(16, 16)] + b1)

            base = f * (DT * NB * 8 * 128) + bt0 * 1024
            for dt in range(DT):
                pltpu.async_copy(
                    outb_v.at[pl.ds(o * GBLK + dt * (G * 8 * 128),
                                    G * 8 * 128)],
                    out_hbm.at[pl.ds(base + dt * (NB * 8 * 128),
                                     G * 8 * 128)],
                    ssems[o])

        def wait_store(o):
            for dt in range(DT):
                pltpu.make_async_copy(
                    outb_v.at[pl.ds(o * GBLK, G * 8 * 128)],
                    out_hbm.at[pl.ds(0, G * 8 * 128)], ssems[o]).wait()

        for k in range(4):
            prep_a(k, k)
        prep_b(0, 0)
        prep_b(1, 1)

        @pl.loop(0, n_groups_w // 4)
        def _main(cc):
            gq = cc * 4
            for j in range(4):
                g = gq + j
                p2 = (j + 2) % 4

                @pl.when(g + 2 < n_groups_w)
                def _():
                    prep_b(g + 2, p2)

                @pl.when(g >= 2)
                def _():
                    wait_store(j % 2)
                proc(g, j, j % 2)

                @pl.when(g + 4 < n_groups_w)
                def _():
                    prep_a(g + 4, j)

        wait_store(0)
        wait_store(1)

    return run


def kernel(inputs, table, bias):
    B, F = inputs.shape
    D = table.shape[1]
    NC, NS = _sc_workers()
    NB = B // 128
    DT = D // 8

    idx_t = inputs.astype(jnp.int32).T.reshape(F, NB, 128)

    run = _build(B, F, D, NC, NS)
    out_flat = run(idx_t, table, bias.astype(jnp.float32))
    out = (out_flat.reshape(F, DT, NB, 8, 128)
           .transpose(2, 4, 0, 1, 3).reshape(B, F, D))
    return out
